# baseline (speedup 1.0000x reference)
import functools

import jax
import jax.numpy as jnp
from jax import lax
from jax.experimental import pallas as pl
from jax.experimental.pallas import tpu as pltpu
from jax.experimental.pallas import tpu_sc as plsc

N = 10000
E = 320000
D = 128
HD = 64
NUM_CLASSES = 25

NC = 2
NS = 16
NW = NC * NS
CSZ = 128
SUP = 2 * CSZ
T_EDGES = 20480
SUPS = T_EDGES // SUP
BULK = 8
NBULKS = SUPS // BULK
EP = NS * T_EDGES
IDXROWS = EP // CSZ
EW_B = EP // NW
CHB = EW_B // CSZ
AGG_ROWS = 10112
ROWS_PER_TILE = AGG_ROWS // NS


@functools.cache
def _sc_mesh():
    return plsc.VectorSubcoreMesh(core_axis_name="c", subcore_axis_name="s",
                                  num_cores=NC, num_subcores=NS)


def _posenc_feats(r, multires):
    feats = [r]
    s, c = jnp.sin(r), jnp.cos(r)
    for _ in range(multires):
        feats.append(s)
        feats.append(c)
        s, c = 2.0 * s * c, 1.0 - 2.0 * s * s
    return jnp.concatenate(feats, axis=0)


def _edge_dist_body(rect_hbm, xi_hbm, xj_hbm, dT_hbm,
                    rect_v, xi_v, xj_v, dbuf):
    wid = lax.axis_index("c") * NS + lax.axis_index("s")
    pltpu.sync_copy(rect_hbm, rect_v)
    pltpu.sync_copy(xi_hbm.at[pl.ds(wid * CHB, CHB)], xi_v)
    pltpu.sync_copy(xj_hbm.at[pl.ds(wid * CHB, CHB)], xj_v)

    def row_body(r, carry):
        for j in range(8):
            s = slice(j * 16, (j + 1) * 16)
            vi = xi_v[r, s] * 4
            vj = xj_v[r, s] * 4
            for c in range(4):
                a = plsc.load_gather(rect_v, [vi + c])
                b = plsc.load_gather(rect_v, [vj + c])
                dbuf[c, pl.ds(r * CSZ + j * 16, 16)] = jnp.abs(a - b)
        return carry

    lax.fori_loop(0, CHB, row_body, 0)
    pltpu.sync_copy(dbuf, dT_hbm.at[:, pl.ds(wid * EW_B, EW_B)])


def _edge_dist(rect_flat, xi_flat, xj_flat):
    return pl.kernel(
        _edge_dist_body,
        out_type=jax.ShapeDtypeStruct((4, EP), jnp.float32),
        mesh=_sc_mesh(),
        scratch_types=[
            pltpu.VMEM((4 * N,), jnp.float32),
            pltpu.VMEM((CHB, CSZ), jnp.int32),
            pltpu.VMEM((CHB, CSZ), jnp.int32),
            pltpu.VMEM((4, EW_B), jnp.float32),
        ],
        compiler_params=pltpu.CompilerParams(needs_layout_passes=False),
    )(rect_flat, xi_flat, xj_flat)


def _msg_pass_body(x2_hbm, e2_hbm, xi_hbm, xj_hbm, plo_hbm, phi_hbm,
                   xi_b, xj_b, xg2, ev2, aggr_sh, gs0, gs1, es0, es1, ss0, ss1):
    core = lax.axis_index("c")
    sub = lax.axis_index("s")
    xoff = core * N
    erow0 = sub * T_EDGES
    irow0 = sub * (SUPS * 2)
    gsems = (gs0, gs1)
    esems = (es0, es1)
    ssems = (ss0, ss1)

    zeros16 = jnp.zeros((16,), jnp.float32)

    def zrow(r, carry):
        for j in range(HD // 16):
            ev2[0, r, j * 16:(j + 1) * 16] = zeros16
        return carry

    lax.fori_loop(0, SUP, zrow, 0)
    base = sub * ROWS_PER_TILE
    pltpu.sync_copy(ev2.at[0], aggr_sh.at[pl.ds(base, SUP)])
    pltpu.sync_copy(ev2.at[0], aggr_sh.at[pl.ds(base + SUP, SUP)])
    pltpu.sync_copy(ev2.at[0, pl.ds(0, ROWS_PER_TILE - 2 * SUP)],
                    aggr_sh.at[pl.ds(base + 2 * SUP, ROWS_PER_TILE - 2 * SUP)])
    plsc.subcore_barrier()

    def _wait_scatter(slot):
        for r in range(2):
            pltpu.make_async_copy(ev2.at[slot, pl.ds(r * CSZ, CSZ)],
                                  aggr_sh.at[xj_b.at[0]], ssems[slot]).wait()

    def _issue(b, k):
        s = b * BULK + k
        slot = k % 2
        if k >= 2:
            _wait_scatter(slot)
        pltpu.async_copy(e2_hbm.at[core, pl.ds(erow0 + s * SUP, SUP)],
                         ev2.at[slot], esems[slot])
        for r in range(2):
            pltpu.async_copy(x2_hbm.at[xi_b.at[2 * k + r]],
                             xg2.at[slot, pl.ds(r * CSZ, CSZ)], gsems[slot])

    def _consume(b, k):
        slot = k % 2
        for r in range(2):
            pltpu.make_async_copy(x2_hbm.at[xi_b.at[2 * k + r]],
                                  xg2.at[slot, pl.ds(r * CSZ, CSZ)],
                                  gsems[slot]).wait()
        pltpu.make_async_copy(e2_hbm.at[core, pl.ds(0, SUP)], ev2.at[slot],
                              esems[slot]).wait()

        @plsc.parallel_loop(0, SUP, 1, unroll=4)
        def crow(r):
            for j in range(HD // 16):
                sl = slice(j * 16, (j + 1) * 16)
                ev2[slot, r, sl] = jnp.maximum(ev2[slot, r, sl] + xg2[slot, r, sl], 0.0)

        for r in range(2):
            pltpu.async_copy(ev2.at[slot, pl.ds(r * CSZ, CSZ)],
                             aggr_sh.at[xj_b.at[2 * k + r]], ssems[slot], add=True)

    def bulk_body(b, carry):
        @pl.when(b > 0)
        def _():
            _wait_scatter(0)
            _wait_scatter(1)

        pltpu.sync_copy(xi_hbm.at[pl.ds(irow0 + b * 2 * BULK, 2 * BULK)], xi_b)
        pltpu.sync_copy(xj_hbm.at[pl.ds(irow0 + b * 2 * BULK, 2 * BULK)], xj_b)

        @pl.when(xoff > 0)
        def _():
            def orow(r, c2):
                for j in range(CSZ // 16):
                    sl = slice(j * 16, (j + 1) * 16)
                    xi_b[r, sl] = xi_b[r, sl] + xoff
                return c2

            lax.fori_loop(0, 2 * BULK, orow, 0)

        _issue(b, 0)
        for k in range(BULK):
            if k + 1 < BULK:
                _issue(b, k + 1)
            _consume(b, k)
        return carry

    lax.fori_loop(0, NBULKS, bulk_body, 0)
    _wait_scatter(0)
    _wait_scatter(1)
    plsc.subcore_barrier()

    @pl.when(core == 0)
    def _():
        pltpu.sync_copy(aggr_sh.at[pl.ds(base, ROWS_PER_TILE)],
                        plo_hbm.at[pl.ds(base, ROWS_PER_TILE)])

    @pl.when(core == 1)
    def _():
        pltpu.sync_copy(aggr_sh.at[pl.ds(base, ROWS_PER_TILE)],
                        phi_hbm.at[pl.ds(base, ROWS_PER_TILE)])


def _msg_pass(x2_flat, e2_flat, xi_flat, xj_flat):
    return pl.kernel(
        _msg_pass_body,
        out_type=(jax.ShapeDtypeStruct((AGG_ROWS, HD), jnp.float32),
                  jax.ShapeDtypeStruct((AGG_ROWS, HD), jnp.float32)),
        mesh=_sc_mesh(),
        scratch_types=[
            pltpu.VMEM((2 * BULK, CSZ), jnp.int32),
            pltpu.VMEM((2 * BULK, CSZ), jnp.int32),
            pltpu.VMEM((2, SUP, HD), jnp.float32),
            pltpu.VMEM((2, SUP, HD), jnp.float32),
            pltpu.VMEM_SHARED((AGG_ROWS, HD), jnp.float32),
            pltpu.SemaphoreType.DMA,
            pltpu.SemaphoreType.DMA,
            pltpu.SemaphoreType.DMA,
            pltpu.SemaphoreType.DMA,
            pltpu.SemaphoreType.DMA,
            pltpu.SemaphoreType.DMA,
        ],
        compiler_params=pltpu.CompilerParams(needs_layout_passes=False,
                                             use_tc_tiling_on_sc=False),
    )(x2_flat, e2_flat, xi_flat, xj_flat)


def _pos_embed_body(rectT_ref, Wp_ref, x0_ref):
    ft = _posenc_feats(rectT_ref[...], 6)
    x0_ref[...] = lax.dot_general(ft, Wp_ref[...], (((0,), (0,)), ((), ())),
                                  preferred_element_type=jnp.float32)


def _pos_embed(rectT, W_pos):
    return pl.pallas_call(
        _pos_embed_body,
        out_shape=jax.ShapeDtypeStruct((N, D), jnp.float32),
    )(rectT, W_pos)


def _embed_body(x0_ref, img_ref, Wi_ref, b_ref, x2_ref):
    img = jnp.dot(img_ref[...], Wi_ref[...], preferred_element_type=jnp.float32)
    val = x0_ref[...] + img + b_ref[...]
    x2_ref[0, :, :] = val[:, :HD]
    x2_ref[1, :, :] = val[:, HD:]


def _embed(x0, images, W_img, b):
    BR = 400
    return pl.pallas_call(
        _embed_body,
        grid=(N // BR,),
        in_specs=[
            pl.BlockSpec((BR, D), lambda i: (i, 0)),
            pl.BlockSpec((BR, 768), lambda i: (i, 0)),
            pl.BlockSpec((768, D), lambda i: (0, 0)),
            pl.BlockSpec((1, D), lambda i: (0, 0)),
        ],
        out_specs=pl.BlockSpec((2, BR, HD), lambda i: (0, i, 0)),
        out_shape=jax.ShapeDtypeStruct((2, N, HD), jnp.float32),
    )(x0, images, W_img, b)


def _edge_mlp_body(dT_ref, W_ref, b_ref, e2_ref):
    ft = _posenc_feats(dT_ref[...], 3)
    e = lax.dot_general(ft, W_ref[...], (((0,), (0,)), ((), ())),
                        preferred_element_type=jnp.float32) + b_ref[...]
    e2_ref[0, :, :] = e[:, :HD]
    e2_ref[1, :, :] = e[:, HD:]


def _edge_mlp(dT, W_edge, b_edge):
    BC = 2048
    return pl.pallas_call(
        _edge_mlp_body,
        grid=(EP // BC,),
        in_specs=[
            pl.BlockSpec((4, BC), lambda i: (0, i)),
            pl.BlockSpec((28, D), lambda i: (0, 0)),
            pl.BlockSpec((1, D), lambda i: (0, 0)),
        ],
        out_specs=pl.BlockSpec((2, BC, HD), lambda i: (0, i, 0)),
        out_shape=jax.ShapeDtypeStruct((2, EP, HD), jnp.float32),
    )(dT, W_edge, b_edge)


def _tail_body(x2_ref, plo_ref, phi_ref, bbox_ref, lab_ref,
               Wg1, bg1, Wg2, bg2, Wc1, bc1, Wc2, bc2, Wl1, bl1, Wl2, bl2,
               logits_ref, pred_ref, tot_ref, cls_ref, reg_ref):
    x = jnp.concatenate([x2_ref[0], x2_ref[1]], axis=-1)
    aggr = jnp.concatenate([plo_ref[:N, :], phi_ref[:N, :]], axis=-1)
    h = jnp.maximum(jnp.dot(x + aggr, Wg1[...], preferred_element_type=jnp.float32)
                    + bg1[...], 0.0)
    gnn = jnp.dot(h, Wg2[...], preferred_element_type=jnp.float32) + bg2[...]
    ch = jnp.maximum(jnp.dot(gnn, Wc1[...], preferred_element_type=jnp.float32)
                     + bc1[...], 0.0)
    logits = jnp.dot(ch, Wc2[...], preferred_element_type=jnp.float32) + bc2[...]
    lh = jnp.maximum(jnp.dot(gnn, Wl1[...], preferred_element_type=jnp.float32)
                     + bl1[...], 0.0)
    pred = jnp.dot(lh, Wl2[...], preferred_element_type=jnp.float32) + bl2[...]
    logits_ref[...] = logits
    pred_ref[...] = pred
    m = jnp.max(logits, axis=-1, keepdims=True)
    lse = jnp.log(jnp.sum(jnp.exp(logits - m), axis=-1, keepdims=True)) + m
    onehot = jax.lax.broadcasted_iota(jnp.int32, (N, NUM_CLASSES), 1) == lab_ref[...]
    picked = jnp.sum(jnp.where(onehot, logits, 0.0), axis=-1, keepdims=True)
    cls = jnp.mean(lse - picked)
    reg = jnp.mean(jnp.abs(pred - bbox_ref[...]))
    cls_ref[...] = cls[None, None]
    reg_ref[...] = reg[None, None]
    tot_ref[...] = (cls + reg)[None, None]


def _tail(x2, plo, phi, bbox, labels2d, Wg1, bg1, Wg2, bg2,
          Wc1, bc1, Wc2, bc2, Wl1, bl1, Wl2, bl2):
    return pl.pallas_call(
        _tail_body,
        out_shape=(
            jax.ShapeDtypeStruct((N, NUM_CLASSES), jnp.float32),
            jax.ShapeDtypeStruct((N, 4), jnp.float32),
            jax.ShapeDtypeStruct((1, 1), jnp.float32),
            jax.ShapeDtypeStruct((1, 1), jnp.float32),
            jax.ShapeDtypeStruct((1, 1), jnp.float32),
        ),
    )(x2, plo, phi, bbox, labels2d, Wg1, bg1, Wg2, bg2,
      Wc1, bc1, Wc2, bc2, Wl1, bl1, Wl2, bl2)


def kernel(images, layer_rect, edges, bbox, labels, node_indices,
           W_pos, b_pos, W_img, b_img, W_edge, b_edge,
           W_g1, b_g1, W_g2, b_g2,
           W_c1, b_c1, W_c2, b_c2,
           W_l1, b_l1, W_l2, b_l2):
    xi = edges[0, :].astype(jnp.int32)
    xj = edges[1, :].astype(jnp.int32)
    pad = EP - E
    xi_flat = jnp.concatenate([xi, jnp.zeros((pad,), jnp.int32)]).reshape(IDXROWS, CSZ)
    xj_flat = jnp.concatenate([xj, jnp.full((pad,), N, jnp.int32)]).reshape(IDXROWS, CSZ)

    rect_flat = layer_rect.reshape(4 * N)

    x0 = _pos_embed(layer_rect.T, W_pos)
    x2 = _embed(x0, images, W_img, (b_pos + b_img).reshape(1, D))
    dT = _edge_dist(rect_flat, xi_flat, xj_flat)
    e2 = _edge_mlp(dT, W_edge, b_edge.reshape(1, D))
    plo, phi = _msg_pass(x2.reshape(2 * N, HD), e2, xi_flat, xj_flat)

    logits, pred, tot, cls, reg = _tail(
        x2, plo, phi, bbox, labels.astype(jnp.int32).reshape(N, 1),
        W_g1, b_g1.reshape(1, D), W_g2, b_g2.reshape(1, D),
        W_c1, b_c1.reshape(1, D), W_c2, b_c2.reshape(1, NUM_CLASSES),
        W_l1, b_l1.reshape(1, D), W_l2, b_l2.reshape(1, 4))
    return (logits, pred, tot.reshape(()), cls.reshape(()), reg.reshape(()))


# speedup vs baseline: 1.0520x; 1.0520x over previous
import functools

import jax
import jax.numpy as jnp
import numpy as np
from jax import lax
from jax.experimental import pallas as pl
from jax.experimental.pallas import tpu as pltpu
from jax.experimental.pallas import tpu_sc as plsc

N = 10000
E = 320000
D = 128
NUM_CLASSES = 25

NC = 2
NS = 16
NW = NC * NS
CSZ = 128
EW = 10240
CH = EW // CSZ
EP = NW * EW
IDXROWS = EP // CSZ
AGG_ROWS = 10112
ROWS_PER_TILE = AGG_ROWS // NS


@functools.cache
def _sc_mesh():
    return plsc.VectorSubcoreMesh(core_axis_name="c", subcore_axis_name="s",
                                  num_cores=NC, num_subcores=NS)


def _posenc_feats(r, multires):
    feats = [r]
    s, c = jnp.sin(r), jnp.cos(r)
    for _ in range(multires):
        feats.append(s)
        feats.append(c)
        s, c = 2.0 * s * c, 1.0 - 2.0 * s * s
    return jnp.concatenate(feats, axis=0)


def _edge_dist_body(rect_hbm, xi_hbm, xj_hbm, dT_hbm,
                    rect_v, xi_v, xj_v, dbuf):
    wid = lax.axis_index("c") * NS + lax.axis_index("s")
    pltpu.sync_copy(rect_hbm, rect_v)
    pltpu.sync_copy(xi_hbm.at[pl.ds(wid * CH, CH)], xi_v)
    pltpu.sync_copy(xj_hbm.at[pl.ds(wid * CH, CH)], xj_v)

    def row_body(r, carry):
        for j in range(8):
            s = slice(j * 16, (j + 1) * 16)
            vi = xi_v[r, s] * 4
            vj = xj_v[r, s] * 4
            for c in range(4):
                a = plsc.load_gather(rect_v, [vi + c])
                b = plsc.load_gather(rect_v, [vj + c])
                dbuf[c, pl.ds(r * CSZ + j * 16, 16)] = jnp.abs(a - b)
        return carry

    lax.fori_loop(0, CH, row_body, 0)
    pltpu.sync_copy(dbuf, dT_hbm.at[:, pl.ds(wid * EW, EW)])


def _edge_dist(rect_flat, xi_flat, xj_flat):
    return pl.kernel(
        _edge_dist_body,
        out_type=jax.ShapeDtypeStruct((4, EP), jnp.float32),
        mesh=_sc_mesh(),
        scratch_types=[
            pltpu.VMEM((4 * N,), jnp.float32),
            pltpu.VMEM((CH, CSZ), jnp.int32),
            pltpu.VMEM((CH, CSZ), jnp.int32),
            pltpu.VMEM((4, EW), jnp.float32),
        ],
        compiler_params=pltpu.CompilerParams(needs_layout_passes=False),
    )(rect_flat, xi_flat, xj_flat)


def _msg_pass_body(x_hbm, e_hbm, ij_hbm, parts_hbm,
                   ij2, xg2, ev, aggr_sh, gs0, gs1, esem, ssem):
    core = lax.axis_index("c")
    sub = lax.axis_index("s")
    wid = core * NS + sub
    gsems = (gs0, gs1)

    zeros16 = jnp.zeros((16,), jnp.float32)

    def zrow(r, carry):
        for j in range(D // 16):
            ev[r, j * 16:(j + 1) * 16] = zeros16
        return carry

    lax.fori_loop(0, CSZ, zrow, 0)
    base = sub * ROWS_PER_TILE
    for q in range(ROWS_PER_TILE // CSZ):
        pltpu.sync_copy(ev, aggr_sh.at[pl.ds(base + q * CSZ, CSZ)])
    rem = ROWS_PER_TILE % CSZ
    if rem:
        pltpu.sync_copy(ev.at[pl.ds(0, rem)],
                        aggr_sh.at[pl.ds(base + ROWS_PER_TILE - rem, rem)])
    plsc.subcore_barrier()

    def _issue(c, slot):
        pltpu.sync_copy(ij_hbm.at[pl.ds((wid * CH + c) * 2, 2)], ij2.at[slot])
        pltpu.async_copy(x_hbm.at[ij2.at[slot, 0]], xg2.at[slot], gsems[slot])

    def _consume(c, slot):
        pltpu.make_async_copy(x_hbm.at[ij2.at[slot, 0]], xg2.at[slot],
                              gsems[slot]).wait()
        pltpu.sync_copy(e_hbm.at[pl.ds(wid * EW + c * CSZ, CSZ)], ev)

        @plsc.parallel_loop(0, CSZ, 1, unroll=2)
        def crow(r):
            for j in range(D // 16):
                sl = slice(j * 16, (j + 1) * 16)
                ev[r, sl] = jnp.maximum(ev[r, sl] + xg2[slot, r, sl], 0.0)

        pltpu.async_copy(ev, aggr_sh.at[ij2.at[slot, 1]], ssem, add=True)

    _issue(0, 0)

    def pair_body(g, carry):
        for b in range(2):
            c = g * 2 + b

            @pl.when(c >= 1)
            def _():
                pltpu.make_async_copy(ev, aggr_sh.at[ij2.at[0, 1]], ssem).wait()

            @pl.when(c + 1 < CH)
            def _():
                _issue(c + 1, (b + 1) % 2)

            _consume(c, b)
        return carry

    lax.fori_loop(0, CH // 2, pair_body, 0)
    pltpu.make_async_copy(ev, aggr_sh.at[ij2.at[0, 1]], ssem).wait()
    plsc.subcore_barrier()
    pltpu.sync_copy(aggr_sh.at[pl.ds(base, ROWS_PER_TILE)],
                    parts_hbm.at[core, pl.ds(base, ROWS_PER_TILE)])


def _msg_pass(x, e, ijrows):
    return pl.kernel(
        _msg_pass_body,
        out_type=jax.ShapeDtypeStruct((NC, AGG_ROWS, D), jnp.float32),
        mesh=_sc_mesh(),
        scratch_types=[
            pltpu.VMEM((2, 2, CSZ), jnp.int32),
            pltpu.VMEM((2, CSZ, D), jnp.float32),
            pltpu.VMEM((CSZ, D), jnp.float32),
            pltpu.VMEM_SHARED((AGG_ROWS, D), jnp.float32),
            pltpu.SemaphoreType.DMA,
            pltpu.SemaphoreType.DMA,
            pltpu.SemaphoreType.DMA,
            pltpu.SemaphoreType.DMA,
        ],
        compiler_params=pltpu.CompilerParams(needs_layout_passes=False),
    )(x, e, ijrows)


def _pos_embed_body(rectT_ref, Wp_ref, x0_ref):
    ft = _posenc_feats(rectT_ref[...], 6)
    x0_ref[...] = lax.dot_general(ft, Wp_ref[...], (((0,), (0,)), ((), ())),
                                  preferred_element_type=jnp.float32)


def _pos_embed(rectT, W_pos):
    return pl.pallas_call(
        _pos_embed_body,
        out_shape=jax.ShapeDtypeStruct((N, D), jnp.float32),
    )(rectT, W_pos)


def _embed_body(x0_ref, img_ref, Wi_ref, b_ref, x_ref):
    img = jnp.dot(img_ref[...], Wi_ref[...], preferred_element_type=jnp.float32)
    x_ref[...] = x0_ref[...] + img + b_ref[...]


def _embed(x0, images, W_img, b):
    BR = 400
    return pl.pallas_call(
        _embed_body,
        grid=(N // BR,),
        in_specs=[
            pl.BlockSpec((BR, D), lambda i: (i, 0)),
            pl.BlockSpec((BR, 768), lambda i: (i, 0)),
            pl.BlockSpec((768, D), lambda i: (0, 0)),
            pl.BlockSpec((1, D), lambda i: (0, 0)),
        ],
        out_specs=pl.BlockSpec((BR, D), lambda i: (i, 0)),
        out_shape=jax.ShapeDtypeStruct((N, D), jnp.float32),
    )(x0, images, W_img, b)


def _edge_mlp_body(dT_ref, W_ref, b_ref, e_ref):
    ft = _posenc_feats(dT_ref[...], 3)
    e_ref[...] = lax.dot_general(ft, W_ref[...], (((0,), (0,)), ((), ())),
                                 preferred_element_type=jnp.float32) + b_ref[...]


def _edge_mlp(dT, W_edge, b_edge):
    BC = 2048
    return pl.pallas_call(
        _edge_mlp_body,
        grid=(EP // BC,),
        in_specs=[
            pl.BlockSpec((4, BC), lambda i: (0, i)),
            pl.BlockSpec((28, D), lambda i: (0, 0)),
            pl.BlockSpec((1, D), lambda i: (0, 0)),
        ],
        out_specs=pl.BlockSpec((BC, D), lambda i: (i, 0)),
        out_shape=jax.ShapeDtypeStruct((EP, D), jnp.float32),
    )(dT, W_edge, b_edge)


def _tail_body(x_ref, parts_ref, bbox_ref, lab_ref,
               Wg1, bg1, Wg2, bg2, Wc1, bc1, Wc2, bc2, Wl1, bl1, Wl2, bl2,
               logits_ref, pred_ref, tot_ref, cls_ref, reg_ref):
    x = x_ref[...]
    aggr = parts_ref[0, :N, :] + parts_ref[1, :N, :]
    h = jnp.maximum(jnp.dot(x + aggr, Wg1[...], preferred_element_type=jnp.float32)
                    + bg1[...], 0.0)
    gnn = jnp.dot(h, Wg2[...], preferred_element_type=jnp.float32) + bg2[...]
    ch = jnp.maximum(jnp.dot(gnn, Wc1[...], preferred_element_type=jnp.float32)
                     + bc1[...], 0.0)
    logits = jnp.dot(ch, Wc2[...], preferred_element_type=jnp.float32) + bc2[...]
    lh = jnp.maximum(jnp.dot(gnn, Wl1[...], preferred_element_type=jnp.float32)
                     + bl1[...], 0.0)
    pred = jnp.dot(lh, Wl2[...], preferred_element_type=jnp.float32) + bl2[...]
    logits_ref[...] = logits
    pred_ref[...] = pred
    m = jnp.max(logits, axis=-1, keepdims=True)
    lse = jnp.log(jnp.sum(jnp.exp(logits - m), axis=-1, keepdims=True)) + m
    onehot = jax.lax.broadcasted_iota(jnp.int32, (N, NUM_CLASSES), 1) == lab_ref[...]
    picked = jnp.sum(jnp.where(onehot, logits, 0.0), axis=-1, keepdims=True)
    cls = jnp.mean(lse - picked)
    reg = jnp.mean(jnp.abs(pred - bbox_ref[...]))
    cls_ref[...] = cls[None, None]
    reg_ref[...] = reg[None, None]
    tot_ref[...] = (cls + reg)[None, None]


def _tail(x, parts, bbox, labels2d, Wg1, bg1, Wg2, bg2,
          Wc1, bc1, Wc2, bc2, Wl1, bl1, Wl2, bl2):
    return pl.pallas_call(
        _tail_body,
        out_shape=(
            jax.ShapeDtypeStruct((N, NUM_CLASSES), jnp.float32),
            jax.ShapeDtypeStruct((N, 4), jnp.float32),
            jax.ShapeDtypeStruct((1, 1), jnp.float32),
            jax.ShapeDtypeStruct((1, 1), jnp.float32),
            jax.ShapeDtypeStruct((1, 1), jnp.float32),
        ),
    )(x, parts, bbox, labels2d, Wg1, bg1, Wg2, bg2,
      Wc1, bc1, Wc2, bc2, Wl1, bl1, Wl2, bl2)


def kernel(images, layer_rect, edges, bbox, labels, node_indices,
           W_pos, b_pos, W_img, b_img, W_edge, b_edge,
           W_g1, b_g1, W_g2, b_g2,
           W_c1, b_c1, W_c2, b_c2,
           W_l1, b_l1, W_l2, b_l2):
    xi = edges[0, :].astype(jnp.int32)
    xj = edges[1, :].astype(jnp.int32)
    pad = EP - E
    xi_flat = jnp.concatenate([xi, jnp.zeros((pad,), jnp.int32)]).reshape(IDXROWS, CSZ)
    xj_flat = jnp.concatenate([xj, jnp.full((pad,), N, jnp.int32)]).reshape(IDXROWS, CSZ)
    ijrows = jnp.stack([xi_flat, xj_flat], axis=1).reshape(2 * IDXROWS, CSZ)

    rect_flat = layer_rect.reshape(4 * N)

    x0 = _pos_embed(layer_rect.T, W_pos)
    x = _embed(x0, images, W_img, (b_pos + b_img).reshape(1, D))
    dT = _edge_dist(rect_flat, xi_flat, xj_flat)
    e = _edge_mlp(dT, W_edge, b_edge.reshape(1, D))
    parts = _msg_pass(x, e, ijrows)

    logits, pred, tot, cls, reg = _tail(
        x, parts, bbox, labels.astype(jnp.int32).reshape(N, 1),
        W_g1, b_g1.reshape(1, D), W_g2, b_g2.reshape(1, D),
        W_c1, b_c1.reshape(1, D), W_c2, b_c2.reshape(1, NUM_CLASSES),
        W_l1, b_l1.reshape(1, D), W_l2, b_l2.reshape(1, 4))
    return (logits, pred, tot.reshape(()), cls.reshape(()), reg.reshape(()))


# speedup vs baseline: 1.0543x; 1.0021x over previous
import functools

import jax
import jax.numpy as jnp
import numpy as np
from jax import lax
from jax.experimental import pallas as pl
from jax.experimental.pallas import tpu as pltpu
from jax.experimental.pallas import tpu_sc as plsc

N = 10000
E = 320000
D = 128
NUM_CLASSES = 25

NC = 2
NS = 16
NW = NC * NS
CSZ = 128
EW = 10240
CH = EW // CSZ
EP = NW * EW
IDXROWS = EP // CSZ
AGG_ROWS = 10112
ROWS_PER_TILE = AGG_ROWS // NS


@functools.cache
def _sc_mesh():
    return plsc.VectorSubcoreMesh(core_axis_name="c", subcore_axis_name="s",
                                  num_cores=NC, num_subcores=NS)


def _posenc_feats(r, multires):
    feats = [r]
    s, c = jnp.sin(r), jnp.cos(r)
    for _ in range(multires):
        feats.append(s)
        feats.append(c)
        s, c = 2.0 * s * c, 1.0 - 2.0 * s * s
    return jnp.concatenate(feats, axis=0)


def _edge_dist_body(rect_hbm, xi_hbm, xj_hbm, dT_hbm,
                    rect_v, xi_v, xj_v, dbuf):
    wid = lax.axis_index("c") * NS + lax.axis_index("s")
    pltpu.sync_copy(rect_hbm, rect_v)
    pltpu.sync_copy(xi_hbm.at[pl.ds(wid * CH, CH)], xi_v)
    pltpu.sync_copy(xj_hbm.at[pl.ds(wid * CH, CH)], xj_v)

    def row_body(r, carry):
        for j in range(8):
            s = slice(j * 16, (j + 1) * 16)
            vi = xi_v[r, s] * 4
            vj = xj_v[r, s] * 4
            for c in range(4):
                a = plsc.load_gather(rect_v, [vi + c])
                b = plsc.load_gather(rect_v, [vj + c])
                dbuf[c, pl.ds(r * CSZ + j * 16, 16)] = jnp.abs(a - b)
        return carry

    lax.fori_loop(0, CH, row_body, 0)
    pltpu.sync_copy(dbuf, dT_hbm.at[:, pl.ds(wid * EW, EW)])


def _edge_dist(rect_flat, xi_flat, xj_flat):
    return pl.kernel(
        _edge_dist_body,
        out_type=jax.ShapeDtypeStruct((4, EP), jnp.float32),
        mesh=_sc_mesh(),
        scratch_types=[
            pltpu.VMEM((4 * N,), jnp.float32),
            pltpu.VMEM((CH, CSZ), jnp.int32),
            pltpu.VMEM((CH, CSZ), jnp.int32),
            pltpu.VMEM((4, EW), jnp.float32),
        ],
        compiler_params=pltpu.CompilerParams(needs_layout_passes=False),
    )(rect_flat, xi_flat, xj_flat)


def _msg_pass_body(x_hbm, e_hbm, ij_hbm, parts_hbm,
                   ij2, xg2, ev, aggr_sh, gs0, gs1, esem, ssem):
    core = lax.axis_index("c")
    sub = lax.axis_index("s")
    wid = sub * NC + core
    gsems = (gs0, gs1)

    zeros16 = jnp.zeros((16,), jnp.float32)

    def zrow(r, carry):
        for j in range(D // 16):
            ev[r, j * 16:(j + 1) * 16] = zeros16
        return carry

    lax.fori_loop(0, CSZ, zrow, 0)
    base = sub * ROWS_PER_TILE
    for q in range(ROWS_PER_TILE // CSZ):
        pltpu.sync_copy(ev, aggr_sh.at[pl.ds(base + q * CSZ, CSZ)])
    rem = ROWS_PER_TILE % CSZ
    if rem:
        pltpu.sync_copy(ev.at[pl.ds(0, rem)],
                        aggr_sh.at[pl.ds(base + ROWS_PER_TILE - rem, rem)])
    plsc.subcore_barrier()

    def _issue(c, slot):
        pltpu.sync_copy(ij_hbm.at[pl.ds((wid * CH + c) * 2, 2)], ij2.at[slot])
        pltpu.async_copy(x_hbm.at[ij2.at[slot, 0]], xg2.at[slot], gsems[slot])

    def _consume(c, slot):
        pltpu.make_async_copy(x_hbm.at[ij2.at[slot, 0]], xg2.at[slot],
                              gsems[slot]).wait()
        pltpu.sync_copy(e_hbm.at[pl.ds(wid * EW + c * CSZ, CSZ)], ev)

        @plsc.parallel_loop(0, CSZ, 1, unroll=2)
        def crow(r):
            for j in range(D // 16):
                sl = slice(j * 16, (j + 1) * 16)
                ev[r, sl] = jnp.maximum(ev[r, sl] + xg2[slot, r, sl], 0.0)

        pltpu.async_copy(ev, aggr_sh.at[ij2.at[slot, 1]], ssem, add=True)

    _issue(0, 0)

    def pair_body(g, carry):
        for b in range(2):
            c = g * 2 + b

            @pl.when(c >= 1)
            def _():
                pltpu.make_async_copy(ev, aggr_sh.at[ij2.at[0, 1]], ssem).wait()

            @pl.when(c + 1 < CH)
            def _():
                _issue(c + 1, (b + 1) % 2)

            _consume(c, b)
        return carry

    lax.fori_loop(0, CH // 2, pair_body, 0)
    pltpu.make_async_copy(ev, aggr_sh.at[ij2.at[0, 1]], ssem).wait()
    plsc.subcore_barrier()
    pltpu.sync_copy(aggr_sh.at[pl.ds(base, ROWS_PER_TILE)],
                    parts_hbm.at[core, pl.ds(base, ROWS_PER_TILE)])


def _msg_pass(x, e, ijrows):
    return pl.kernel(
        _msg_pass_body,
        out_type=jax.ShapeDtypeStruct((NC, AGG_ROWS, D), jnp.float32),
        mesh=_sc_mesh(),
        scratch_types=[
            pltpu.VMEM((2, 2, CSZ), jnp.int32),
            pltpu.VMEM((2, CSZ, D), jnp.float32),
            pltpu.VMEM((CSZ, D), jnp.float32),
            pltpu.VMEM_SHARED((AGG_ROWS, D), jnp.float32),
            pltpu.SemaphoreType.DMA,
            pltpu.SemaphoreType.DMA,
            pltpu.SemaphoreType.DMA,
            pltpu.SemaphoreType.DMA,
        ],
        compiler_params=pltpu.CompilerParams(needs_layout_passes=False),
    )(x, e, ijrows)


def _pos_embed_body(rectT_ref, Wp_ref, x0_ref):
    ft = _posenc_feats(rectT_ref[...], 6)
    x0_ref[...] = lax.dot_general(ft, Wp_ref[...], (((0,), (0,)), ((), ())),
                                  preferred_element_type=jnp.float32)


def _pos_embed(rectT, W_pos):
    return pl.pallas_call(
        _pos_embed_body,
        out_shape=jax.ShapeDtypeStruct((N, D), jnp.float32),
    )(rectT, W_pos)


def _embed_body(x0_ref, img_ref, Wi_ref, b_ref, x_ref):
    img = jnp.dot(img_ref[...], Wi_ref[...], preferred_element_type=jnp.float32)
    x_ref[...] = x0_ref[...] + img + b_ref[...]


def _embed(x0, images, W_img, b):
    BR = 400
    return pl.pallas_call(
        _embed_body,
        grid=(N // BR,),
        in_specs=[
            pl.BlockSpec((BR, D), lambda i: (i, 0)),
            pl.BlockSpec((BR, 768), lambda i: (i, 0)),
            pl.BlockSpec((768, D), lambda i: (0, 0)),
            pl.BlockSpec((1, D), lambda i: (0, 0)),
        ],
        out_specs=pl.BlockSpec((BR, D), lambda i: (i, 0)),
        out_shape=jax.ShapeDtypeStruct((N, D), jnp.float32),
    )(x0, images, W_img, b)


def _edge_mlp_body(dT_ref, W_ref, b_ref, e_ref):
    ft = _posenc_feats(dT_ref[...], 3)
    e_ref[...] = lax.dot_general(ft, W_ref[...], (((0,), (0,)), ((), ())),
                                 preferred_element_type=jnp.float32) + b_ref[...]


def _edge_mlp(dT, W_edge, b_edge):
    BC = 2048
    return pl.pallas_call(
        _edge_mlp_body,
        grid=(EP // BC,),
        in_specs=[
            pl.BlockSpec((4, BC), lambda i: (0, i)),
            pl.BlockSpec((28, D), lambda i: (0, 0)),
            pl.BlockSpec((1, D), lambda i: (0, 0)),
        ],
        out_specs=pl.BlockSpec((BC, D), lambda i: (i, 0)),
        out_shape=jax.ShapeDtypeStruct((EP, D), jnp.float32),
    )(dT, W_edge, b_edge)


def _tail_body(x_ref, parts_ref, bbox_ref, lab_ref,
               Wg1, bg1, Wg2, bg2, Wc1, bc1, Wc2, bc2, Wl1, bl1, Wl2, bl2,
               logits_ref, pred_ref, tot_ref, cls_ref, reg_ref):
    x = x_ref[...]
    aggr = parts_ref[0, :N, :] + parts_ref[1, :N, :]
    h = jnp.maximum(jnp.dot(x + aggr, Wg1[...], preferred_element_type=jnp.float32)
                    + bg1[...], 0.0)
    gnn = jnp.dot(h, Wg2[...], preferred_element_type=jnp.float32) + bg2[...]
    ch = jnp.maximum(jnp.dot(gnn, Wc1[...], preferred_element_type=jnp.float32)
                     + bc1[...], 0.0)
    logits = jnp.dot(ch, Wc2[...], preferred_element_type=jnp.float32) + bc2[...]
    lh = jnp.maximum(jnp.dot(gnn, Wl1[...], preferred_element_type=jnp.float32)
                     + bl1[...], 0.0)
    pred = jnp.dot(lh, Wl2[...], preferred_element_type=jnp.float32) + bl2[...]
    logits_ref[...] = logits
    pred_ref[...] = pred
    m = jnp.max(logits, axis=-1, keepdims=True)
    lse = jnp.log(jnp.sum(jnp.exp(logits - m), axis=-1, keepdims=True)) + m
    onehot = jax.lax.broadcasted_iota(jnp.int32, (N, NUM_CLASSES), 1) == lab_ref[...]
    picked = jnp.sum(jnp.where(onehot, logits, 0.0), axis=-1, keepdims=True)
    cls = jnp.mean(lse - picked)
    reg = jnp.mean(jnp.abs(pred - bbox_ref[...]))
    cls_ref[...] = cls[None, None]
    reg_ref[...] = reg[None, None]
    tot_ref[...] = (cls + reg)[None, None]


def _tail(x, parts, bbox, labels2d, Wg1, bg1, Wg2, bg2,
          Wc1, bc1, Wc2, bc2, Wl1, bl1, Wl2, bl2):
    return pl.pallas_call(
        _tail_body,
        out_shape=(
            jax.ShapeDtypeStruct((N, NUM_CLASSES), jnp.float32),
            jax.ShapeDtypeStruct((N, 4), jnp.float32),
            jax.ShapeDtypeStruct((1, 1), jnp.float32),
            jax.ShapeDtypeStruct((1, 1), jnp.float32),
            jax.ShapeDtypeStruct((1, 1), jnp.float32),
        ),
    )(x, parts, bbox, labels2d, Wg1, bg1, Wg2, bg2,
      Wc1, bc1, Wc2, bc2, Wl1, bl1, Wl2, bl2)


def kernel(images, layer_rect, edges, bbox, labels, node_indices,
           W_pos, b_pos, W_img, b_img, W_edge, b_edge,
           W_g1, b_g1, W_g2, b_g2,
           W_c1, b_c1, W_c2, b_c2,
           W_l1, b_l1, W_l2, b_l2):
    xi = edges[0, :].astype(jnp.int32)
    xj = edges[1, :].astype(jnp.int32)
    pad = EP - E
    xi_flat = jnp.concatenate([xi, jnp.zeros((pad,), jnp.int32)]).reshape(IDXROWS, CSZ)
    xj_flat = jnp.concatenate([xj, jnp.full((pad,), N, jnp.int32)]).reshape(IDXROWS, CSZ)
    ijrows = jnp.stack([xi_flat, xj_flat], axis=1).reshape(2 * IDXROWS, CSZ)

    rect_flat = layer_rect.reshape(4 * N)

    x0 = _pos_embed(layer_rect.T, W_pos)
    x = _embed(x0, images, W_img, (b_pos + b_img).reshape(1, D))
    dT = _edge_dist(rect_flat, xi_flat, xj_flat)
    e = _edge_mlp(dT, W_edge, b_edge.reshape(1, D))
    parts = _msg_pass(x, e, ijrows)

    logits, pred, tot, cls, reg = _tail(
        x, parts, bbox, labels.astype(jnp.int32).reshape(N, 1),
        W_g1, b_g1.reshape(1, D), W_g2, b_g2.reshape(1, D),
        W_c1, b_c1.reshape(1, D), W_c2, b_c2.reshape(1, NUM_CLASSES),
        W_l1, b_l1.reshape(1, D), W_l2, b_l2.reshape(1, 4))
    return (logits, pred, tot.reshape(()), cls.reshape(()), reg.reshape(()))
